# baseline (speedup 1.0000x reference)
import functools

import jax
import jax.numpy as jnp
from jax import lax
from jax.experimental import pallas as pl
from jax.experimental.pallas import tpu as pltpu
from jax.experimental.pallas import tpu_sc as plsc

_NC = 2
_NS = 16


def kernel(x, W):
    B, P, D = x.shape
    nw = _NC * _NS
    ng = nw // 2
    nb = B // ng
    Dh = D // 2
    Wt = jnp.swapaxes(W, 0, 1)
    mesh = plsc.VectorSubcoreMesh(core_axis_name="c", subcore_axis_name="s")

    @functools.partial(
        pl.kernel,
        out_type=jax.ShapeDtypeStruct((B, D, P), W.dtype),
        mesh=mesh,
        scratch_types=[
            pltpu.VMEM((Dh, P), W.dtype),
            pltpu.SemaphoreType.DMA,
        ],
    )
    def sc_broadcast(w_hbm, out_hbm, wbuf, sem):
        c = lax.axis_index("c")
        s = lax.axis_index("s")
        wid = c * _NS + s
        half = wid % 2
        group = wid // 2
        pltpu.sync_copy(w_hbm.at[pl.ds(half * Dh, Dh)], wbuf)
        base = group * nb
        copies = [
            pltpu.make_async_copy(
                wbuf, out_hbm.at[base + i, pl.ds(half * Dh, Dh)], sem
            )
            for i in range(nb)
        ]
        for cp in copies:
            cp.start()
        for cp in copies:
            cp.wait()

    out_t = sc_broadcast(Wt)
    return jnp.swapaxes(out_t, 1, 2)
